# baseline (speedup 1.0000x reference)
import jax
import jax.numpy as jnp
from jax import lax
from jax.experimental import pallas as pl
from jax.experimental.pallas import tpu as pltpu
from jax.experimental.pallas import tpu_sc as plsc

N = 10000
NP = 10240
E = 320000
D = 128

NC = 2
NS = 16
NW = NC * NS
EPW = E // NW
EPP = 10240
CH = 64
NCHUNK = EPP // CH
SC2 = 128
NSUP = EPP // SC2
NB = 4
NI = 4
RPT = NP // NS
WR = NP // D


def _scale_chunk(buf, wsum_loc, ring, sj, h, iota16):
    def grp(g, _):
        d16 = ring[sj, 0, pl.ds(64 * h + 16 * g, 16)]
        w16 = lax.bitcast_convert_type(
            ring[sj, 2, pl.ds(64 * h + 16 * g, 16)], jnp.float32)
        for r in range(16):
            w = w16[r]
            d = d16[r]
            i = 16 * g + r
            for j in range(D // 16):
                buf[i, pl.ds(16 * j, 16)] = buf[i, pl.ds(16 * j, 16)] * w
            row = lax.shift_right_logical(d, 7)
            colg = lax.shift_right_logical(d, 4) & 7
            onehot = jnp.where(iota16 == (d & 15), w, 0.0)
            plsc.addupdate(wsum_loc.at[row, pl.ds(colg * 16, 16)], onehot)
        return 0
    lax.fori_loop(0, CH // 16, grp, 0)


def _sc_body(x_hbm, ei_hbm, wbt_hbm, out0_hbm, out1_hbm, outw_hbm,
             ring, b0, b1, b2, b3, wsum_loc, agg_sh,
             gsem0, gsem1, gsem2, gsem3, gsem4, gsem5, gsem6, gsem7,
             ssem0, ssem1, ssem2, ssem3,
             isem0, isem1, isem2, isem3, zsem):
    c = lax.axis_index("c")
    s = lax.axis_index("s")
    wid = s * NC + c
    jbase = wid * NSUP

    bufs = (b0, b1, b2, b3)
    gsems = ((gsem0, gsem1), (gsem2, gsem3), (gsem4, gsem5), (gsem6, gsem7))
    ssems = (ssem0, ssem1, ssem2, ssem3)
    isems = (isem0, isem1, isem2, isem3)

    def gather_issue(slot, h, b):
        pltpu.async_copy(x_hbm.at[ring.at[slot, 1, pl.ds(64 * h, 32)]],
                         bufs[b].at[pl.ds(0, 32)], gsems[b][0])
        pltpu.async_copy(x_hbm.at[ring.at[slot, 1, pl.ds(64 * h + 32, 32)]],
                         bufs[b].at[pl.ds(32, 32)], gsems[b][1])

    def gather_wait(slot, h, b):
        pltpu.make_async_copy(x_hbm.at[ring.at[slot, 1, pl.ds(64 * h, 32)]],
                              bufs[b].at[pl.ds(0, 32)], gsems[b][0]).wait()
        pltpu.make_async_copy(
            x_hbm.at[ring.at[slot, 1, pl.ds(64 * h + 32, 32)]],
            bufs[b].at[pl.ds(32, 32)], gsems[b][1]).wait()

    def load_sup(j, slot):
        pltpu.async_copy(ei_hbm.at[1, jbase + j], ring.at[slot, 0],
                         isems[slot])
        pltpu.async_copy(ei_hbm.at[0, jbase + j], ring.at[slot, 1],
                         isems[slot])
        pltpu.async_copy(wbt_hbm.at[jbase + j], ring.at[slot, 2], isems[slot])

    def wait_sup(j, slot):
        pltpu.make_async_copy(ei_hbm.at[1, jbase + j], ring.at[slot, 0],
                              isems[slot]).wait()
        pltpu.make_async_copy(ei_hbm.at[0, jbase + j], ring.at[slot, 1],
                              isems[slot]).wait()
        pltpu.make_async_copy(wbt_hbm.at[jbase + j], ring.at[slot, 2],
                              isems[slot]).wait()

    for j in range(2):
        load_sup(j, j)

    def zwrow(i, _):
        for j in range(D // 16):
            wsum_loc[i, pl.ds(16 * j, 16)] = jnp.zeros((16,), jnp.float32)
        return 0
    lax.fori_loop(0, WR, zwrow, 0)
    for k in range(RPT // WR):
        pltpu.async_copy(wsum_loc, agg_sh.at[pl.ds(s * RPT + k * WR, WR)],
                         zsem)
    for k in range(RPT // WR):
        pltpu.make_async_copy(wsum_loc,
                              agg_sh.at[pl.ds(s * RPT + k * WR, WR)],
                              zsem).wait()

    def idx_of(k_slot, h):
        return ring.at[k_slot, 0, pl.ds(64 * h, CH)]

    def src_of(k_slot, h):
        return ring.at[k_slot, 1, pl.ds(64 * h, CH)]

    wait_sup(0, 0)
    gather_issue(0, 0, 0)
    gather_issue(0, 1, 1)

    plsc.subcore_barrier()

    iota16 = lax.broadcasted_iota(jnp.int32, (16,), 0)

    def step(t, _):
        for u in range(8):
            k = 8 * t + u
            b = u % NB
            sj = (u >> 1) % NI
            h = u & 1
            bp2 = (u + 2) % NB
            sj2 = ((u + 2) >> 1) % NI
            h2 = (u + 2) & 1

            @pl.when(k >= 2)
            def _():
                pltpu.make_async_copy(
                    bufs[bp2], agg_sh.at[idx_of(sj2, h2)], ssems[bp2]).wait()

            if h == 0:
                @pl.when((k >> 1) + 2 < NSUP)
                def _():
                    load_sup((k >> 1) + 2, ((u >> 1) + 2) % NI)

            @pl.when(k + 2 < NCHUNK)
            def _():
                if h2 == 0:
                    wait_sup((k >> 1) + 1, sj2)
                gather_issue(sj2, h2, bp2)

            gather_wait(sj, h, b)
            _scale_chunk(bufs[b], wsum_loc, ring, sj, h, iota16)
            pltpu.async_copy(bufs[b], agg_sh.at[idx_of(sj, h)], ssems[b],
                             add=True)
        return 0
    lax.fori_loop(0, NCHUNK // 8, step, 0)

    pltpu.make_async_copy(b2, agg_sh.at[idx_of(3, 0)], ssems[2]).wait()
    pltpu.make_async_copy(b3, agg_sh.at[idx_of(3, 1)], ssems[3]).wait()

    plsc.subcore_barrier()

    @pl.when(c == 0)
    def _():
        pltpu.sync_copy(agg_sh.at[pl.ds(s * RPT, RPT)],
                        out0_hbm.at[pl.ds(s * RPT, RPT)])

    @pl.when(c == 1)
    def _():
        pltpu.sync_copy(agg_sh.at[pl.ds(s * RPT, RPT)],
                        out1_hbm.at[pl.ds(s * RPT, RPT)])

    pltpu.sync_copy(wsum_loc, outw_hbm.at[pl.ds(wid * WR, WR)])


@jax.jit
def _sc_aggregate(x, ei3, wbt2):
    mesh = plsc.VectorSubcoreMesh(core_axis_name="c", subcore_axis_name="s")
    f = pl.kernel(
        _sc_body,
        out_type=(jax.ShapeDtypeStruct((NP, D), jnp.float32),
                  jax.ShapeDtypeStruct((NP, D), jnp.float32),
                  jax.ShapeDtypeStruct((NW * WR, D), jnp.float32)),
        mesh=mesh,
        scratch_types=[
            pltpu.VMEM((NI, 3, SC2), jnp.int32),
            pltpu.VMEM((CH, D), jnp.float32),
            pltpu.VMEM((CH, D), jnp.float32),
            pltpu.VMEM((CH, D), jnp.float32),
            pltpu.VMEM((CH, D), jnp.float32),
            pltpu.VMEM((WR, D), jnp.float32),
            pltpu.VMEM_SHARED((NP, D), jnp.float32),
        ] + [pltpu.SemaphoreType.DMA] * 17,
    )
    return f(x, ei3, wbt2)


def _tc_self_body(x_ref, ws_ref, b_ref, s_ref):
    s_ref[...] = lax.dot_general(
        x_ref[...], ws_ref[...], (((1,), (1,)), ((), ())),
        preferred_element_type=jnp.float32) + b_ref[...]


@jax.jit
def _tc_self(x, wst, bias2d):
    R = 1024
    return pl.pallas_call(
        _tc_self_body,
        grid=(NP // R,),
        in_specs=[
            pl.BlockSpec((R, D), lambda i: (i, 0)),
            pl.BlockSpec((D, D), lambda i: (0, 0)),
            pl.BlockSpec((1, D), lambda i: (0, 0)),
        ],
        out_specs=pl.BlockSpec((R, D), lambda i: (i, 0)),
        out_shape=jax.ShapeDtypeStruct((N, D), jnp.float32),
    )(x, wst, bias2d)


def _tc_body(s_ref, p0_ref, p1_ref, w_ref, wn_ref, o_ref):
    agg = p0_ref[...] + p1_ref[...]
    wsum = jnp.maximum(jnp.sum(w_ref[...], axis=0), 1e-8)
    rows = lax.broadcasted_iota(jnp.int32, (wsum.shape[0] * D, wsum.shape[0]),
                                0)
    cols8 = lax.broadcasted_iota(jnp.int32,
                                 (wsum.shape[0] * D, wsum.shape[0]), 1)
    e8 = jnp.where((rows >> 7) == cols8, 1.0, 0.0)
    t = lax.dot_general(e8, wsum, (((1,), (0,)), ((), ())),
                        precision=lax.Precision.HIGHEST,
                        preferred_element_type=jnp.float32)
    rid = lax.broadcasted_iota(jnp.int32, t.shape, 0)
    cid = lax.broadcasted_iota(jnp.int32, t.shape, 1)
    wcol = jnp.sum(jnp.where((rid & 127) == cid, t, 0.0), axis=-1,
                   keepdims=True)
    neigh = agg / wcol
    out = s_ref[...] + lax.dot_general(
        neigh, wn_ref[...], (((1,), (1,)), ((), ())),
        preferred_element_type=jnp.float32)
    n2 = jnp.sum(out * out, axis=-1, keepdims=True)
    o_ref[...] = out * lax.rsqrt(jnp.maximum(n2, 1e-24))


@jax.jit
def _tc_finish(sself, p0, p1, wparts, wn):
    R = 1024
    return pl.pallas_call(
        _tc_body,
        grid=(NP // R,),
        in_specs=[
            pl.BlockSpec((R, D), lambda i: (i, 0)),
            pl.BlockSpec((R, D), lambda i: (i, 0)),
            pl.BlockSpec((R, D), lambda i: (i, 0)),
            pl.BlockSpec((NW, R // D, D), lambda i: (0, i, 0)),
            pl.BlockSpec((D, D), lambda i: (0, 0)),
        ],
        out_specs=pl.BlockSpec((R, D), lambda i: (i, 0)),
        out_shape=jax.ShapeDtypeStruct((N, D), jnp.float32),
    )(sself, p0, p1, wparts, wn)


def kernel(x, edge_index, edge_weight, W_self, W_neigh, bias):
    epad = NW * EPP - E
    ei3 = jnp.pad(edge_index, ((0, 0), (0, epad))).reshape(2, NW * NSUP, SC2)
    wbt2 = jnp.pad(edge_weight,
                   (0, epad)).view(jnp.int32).reshape(NW * NSUP, SC2)
    sself = _tc_self(x, W_self, bias.reshape(1, D))
    p0, p1, wflat = _sc_aggregate(x, ei3, wbt2)
    wparts = wflat.reshape(NW, WR, D)
    return _tc_finish(sself, p0, p1, wparts, W_neigh)
